# baseline (speedup 1.0000x reference)
import functools

import jax
import jax.numpy as jnp
from jax import lax
from jax.experimental import pallas as pl
from jax.experimental.pallas import tpu as pltpu
from jax.experimental.pallas import tpu_sc as plsc

VOCAB = 100000
HIDDEN = 64
TOKENS = 256
BLK = 8192


def _sc_gather(table, ids):
    info = plsc.get_sparse_core_info()
    nc, ns = info.num_cores, info.num_subcores
    nw = nc * ns
    b_per_w = TOKENS // nw
    mesh = plsc.VectorSubcoreMesh(core_axis_name="c", subcore_axis_name="s")

    @functools.partial(
        pl.kernel,
        mesh=mesh,
        out_type=jax.ShapeDtypeStruct((TOKENS, HIDDEN), jnp.bfloat16),
        scratch_types=[
            pltpu.VMEM((b_per_w,), jnp.int32),
            pltpu.VMEM((b_per_w, HIDDEN), jnp.bfloat16),
            pltpu.SemaphoreType.DMA,
        ],
        compiler_params=pltpu.CompilerParams(use_tc_tiling_on_sc=False),
    )
    def gather_kernel(table_hbm, idx_hbm, out_hbm, idx_v, rows_v, sem):
        wid = lax.axis_index("s") * nc + lax.axis_index("c")
        base = wid * b_per_w
        pltpu.sync_copy(idx_hbm.at[pl.ds(base, b_per_w)], idx_v)
        pltpu.async_copy(table_hbm.at[idx_v], rows_v, sem).wait()
        pltpu.sync_copy(rows_v, out_hbm.at[pl.ds(base, b_per_w)])

    return gather_kernel(table, ids)


def _matmul_body(h_ref, w_ref, out_ref):
    out_ref[...] = lax.dot_general(
        h_ref[...].astype(jnp.float32), w_ref[...],
        dimension_numbers=(((1,), (0,)), ((), ())),
        preferred_element_type=jnp.float32,
    )


def _tc_logits(h, head_w_t):
    grid = pl.cdiv(VOCAB, BLK)
    return pl.pallas_call(
        _matmul_body,
        grid=(grid,),
        in_specs=[
            pl.BlockSpec((TOKENS, HIDDEN), lambda i: (0, 0)),
            pl.BlockSpec((HIDDEN, BLK), lambda i: (0, i)),
        ],
        out_specs=pl.BlockSpec((TOKENS, BLK), lambda i: (0, i)),
        out_shape=jax.ShapeDtypeStruct((TOKENS, VOCAB), jnp.float32),
    )(h, head_w_t)


def kernel(input_ids, embed, head_w):
    b, l = input_ids.shape
    ids_flat = input_ids.reshape(-1).astype(jnp.int32)
    h = _sc_gather(embed.astype(jnp.bfloat16), ids_flat)
    logits = _tc_logits(h, head_w.T)
    return logits.reshape(b, l, VOCAB)


# speedup vs baseline: 2.3867x; 2.3867x over previous
import functools

import jax
import jax.numpy as jnp
from jax import lax
from jax.experimental import pallas as pl
from jax.experimental.pallas import tpu as pltpu
from jax.experimental.pallas import tpu_sc as plsc

VOCAB = 100000
HIDDEN = 64
TOKENS = 256
BLK = 8192
LANE = 128


def _sc_gather(table_t, ids):
    info = plsc.get_sparse_core_info()
    nc, ns = info.num_cores, info.num_subcores
    nw = nc * ns
    b_per_w = TOKENS // nw
    lanes = info.num_lanes
    mesh = plsc.VectorSubcoreMesh(core_axis_name="c", subcore_axis_name="s")

    @functools.partial(
        pl.kernel,
        mesh=mesh,
        out_type=jax.ShapeDtypeStruct((TOKENS, HIDDEN), jnp.float32),
        scratch_types=[
            pltpu.VMEM((lanes,), jnp.int32),
            pltpu.VMEM((b_per_w, HIDDEN, LANE), jnp.float32),
            pltpu.VMEM((b_per_w, HIDDEN), jnp.float32),
            pltpu.SemaphoreType.DMA,
        ],
        compiler_params=pltpu.CompilerParams(needs_layout_passes=False),
    )
    def gather_kernel(table_hbm, idx_hbm, out_hbm, idx_v, slab_v, rows_v, sem):
        wid = lax.axis_index("s") * nc + lax.axis_index("c")
        base = wid * b_per_w
        pltpu.sync_copy(idx_hbm.at[pl.ds(base * 2, lanes)], idx_v)
        ids_vec = idx_v[...]
        lane_ids = lax.iota(jnp.int32, lanes)
        copies = []
        cols = []
        for t in range(b_per_w):
            tok = jnp.max(jnp.where(lane_ids == 2 * t, ids_vec, 0))
            col = pl.multiple_of((tok // LANE) * LANE, LANE)
            cols.append(tok - col)
            copies.append(
                pltpu.async_copy(
                    table_hbm.at[:, pl.ds(col, LANE)], slab_v.at[t], sem
                )
            )
        for t in range(b_per_w):
            copies[t].wait()
            lane_t = cols[t]
            for c in range(HIDDEN // lanes):
                d_idx = c * lanes + lane_ids
                vals = plsc.load_gather(
                    slab_v.at[t], [d_idx, jnp.full((lanes,), 0, jnp.int32) + lane_t]
                )
                rows_v[t, pl.ds(c * lanes, lanes)] = vals
        pltpu.sync_copy(rows_v, out_hbm.at[pl.ds(base, b_per_w)])

    ids2 = jnp.stack([ids, ids], axis=1).reshape(-1)
    return gather_kernel(table_t, ids2)


def _matmul_body(h_ref, w_ref, out_ref):
    out_ref[...] = lax.dot_general(
        h_ref[...], w_ref[...],
        dimension_numbers=(((1,), (0,)), ((), ())),
        preferred_element_type=jnp.float32,
    )


def _tc_logits(h, head_w_t):
    grid = pl.cdiv(VOCAB, BLK)
    return pl.pallas_call(
        _matmul_body,
        grid=(grid,),
        in_specs=[
            pl.BlockSpec((TOKENS, HIDDEN), lambda i: (0, 0)),
            pl.BlockSpec((HIDDEN, BLK), lambda i: (0, i)),
        ],
        out_specs=pl.BlockSpec((TOKENS, BLK), lambda i: (0, i)),
        out_shape=jax.ShapeDtypeStruct((TOKENS, VOCAB), jnp.float32),
    )(h, head_w_t)


def kernel(input_ids, embed, head_w):
    b, l = input_ids.shape
    ids_flat = input_ids.reshape(-1).astype(jnp.int32)
    h = _sc_gather(embed.T, ids_flat)
    logits = _tc_logits(h, head_w.T)
    return logits.reshape(b, l, VOCAB)


# speedup vs baseline: 2.4093x; 1.0095x over previous
import functools

import jax
import jax.numpy as jnp
from jax import lax
from jax.experimental import pallas as pl
from jax.experimental.pallas import tpu as pltpu
from jax.experimental.pallas import tpu_sc as plsc

VOCAB = 100000
HIDDEN = 64
TOKENS = 256
BLK = 12800
LANE = 128


def _sc_gather(table_t, ids):
    info = plsc.get_sparse_core_info()
    nc, ns = info.num_cores, info.num_subcores
    nw = nc * ns
    b_per_w = TOKENS // nw
    lanes = info.num_lanes
    mesh = plsc.VectorSubcoreMesh(core_axis_name="c", subcore_axis_name="s")

    @functools.partial(
        pl.kernel,
        mesh=mesh,
        out_type=jax.ShapeDtypeStruct((TOKENS, HIDDEN), jnp.float32),
        scratch_types=[
            pltpu.VMEM((lanes,), jnp.int32),
            pltpu.VMEM((b_per_w, HIDDEN, LANE), jnp.float32),
            pltpu.VMEM((b_per_w, HIDDEN), jnp.float32),
            pltpu.SemaphoreType.DMA,
        ],
        compiler_params=pltpu.CompilerParams(needs_layout_passes=False),
    )
    def gather_kernel(table_hbm, idx_hbm, out_hbm, idx_v, slab_v, rows_v, sem):
        wid = lax.axis_index("s") * nc + lax.axis_index("c")
        base = wid * b_per_w
        pltpu.sync_copy(idx_hbm.at[pl.ds(base, b_per_w)], idx_v.at[pl.ds(0, b_per_w)])
        ids_vec = idx_v[...]
        lane_ids = lax.iota(jnp.int32, lanes)
        copies = []
        cols = []
        for t in range(b_per_w):
            tok = jnp.max(jnp.where(lane_ids == t, ids_vec, 0))
            col = pl.multiple_of((tok // LANE) * LANE, LANE)
            cols.append(tok - col)
            copies.append(
                pltpu.async_copy(
                    table_hbm.at[:, pl.ds(col, LANE)], slab_v.at[t], sem
                )
            )
        for t in range(b_per_w):
            copies[t].wait()
            lane_t = cols[t]
            for c in range(HIDDEN // lanes):
                d_idx = c * lanes + lane_ids
                vals = plsc.load_gather(
                    slab_v.at[t], [d_idx, jnp.full((lanes,), 0, jnp.int32) + lane_t]
                )
                rows_v[t, pl.ds(c * lanes, lanes)] = vals
        pltpu.sync_copy(rows_v, out_hbm.at[pl.ds(base, b_per_w)])

    return gather_kernel(table_t, ids)


def _matmul_body(h_ref, w_ref, out_ref):
    out_ref[...] = lax.dot_general(
        h_ref[...], w_ref[...],
        dimension_numbers=(((1,), (0,)), ((), ())),
        preferred_element_type=jnp.float32,
    )


def _tc_logits(h, head_w_t):
    grid = pl.cdiv(VOCAB, BLK)
    return pl.pallas_call(
        _matmul_body,
        grid=(grid,),
        in_specs=[
            pl.BlockSpec((TOKENS, HIDDEN), lambda i: (0, 0)),
            pl.BlockSpec((HIDDEN, BLK), lambda i: (0, i)),
        ],
        out_specs=pl.BlockSpec((TOKENS, BLK), lambda i: (0, i)),
        out_shape=jax.ShapeDtypeStruct((TOKENS, VOCAB), jnp.float32),
    )(h, head_w_t)


def kernel(input_ids, embed, head_w):
    b, l = input_ids.shape
    ids_flat = input_ids.reshape(-1).astype(jnp.int32)
    h = _sc_gather(embed.T, ids_flat)
    logits = _tc_logits(h, head_w.T)
    return logits.reshape(b, l, VOCAB)


# speedup vs baseline: 2.4112x; 1.0008x over previous
import functools

import jax
import jax.numpy as jnp
from jax import lax
from jax.experimental import pallas as pl
from jax.experimental.pallas import tpu as pltpu
from jax.experimental.pallas import tpu_sc as plsc

VOCAB = 100000
HIDDEN = 64
TOKENS = 256
BLK = 20480
LANE = 128


def _sc_gather(table_t, ids):
    info = plsc.get_sparse_core_info()
    nc, ns = info.num_cores, info.num_subcores
    nw = nc * ns
    b_per_w = TOKENS // nw
    lanes = info.num_lanes
    mesh = plsc.VectorSubcoreMesh(core_axis_name="c", subcore_axis_name="s")

    @functools.partial(
        pl.kernel,
        mesh=mesh,
        out_type=jax.ShapeDtypeStruct((TOKENS, HIDDEN), jnp.float32),
        scratch_types=[
            pltpu.VMEM((lanes,), jnp.int32),
            pltpu.VMEM((b_per_w, HIDDEN, LANE), jnp.float32),
            pltpu.VMEM((b_per_w, HIDDEN), jnp.float32),
            pltpu.SemaphoreType.DMA,
        ],
        compiler_params=pltpu.CompilerParams(needs_layout_passes=False),
    )
    def gather_kernel(table_hbm, idx_hbm, out_hbm, idx_v, slab_v, rows_v, sem):
        wid = lax.axis_index("s") * nc + lax.axis_index("c")
        base = wid * b_per_w
        pltpu.sync_copy(idx_hbm.at[pl.ds(base, b_per_w)], idx_v.at[pl.ds(0, b_per_w)])
        ids_vec = idx_v[...]
        lane_ids = lax.iota(jnp.int32, lanes)
        copies = []
        cols = []
        for t in range(b_per_w):
            tok = jnp.max(jnp.where(lane_ids == t, ids_vec, 0))
            col = pl.multiple_of((tok // LANE) * LANE, LANE)
            cols.append(tok - col)
            copies.append(
                pltpu.async_copy(
                    table_hbm.at[:, pl.ds(col, LANE)], slab_v.at[t], sem
                )
            )
        for t in range(b_per_w):
            copies[t].wait()
            lane_t = cols[t]
            for c in range(HIDDEN // lanes):
                d_idx = c * lanes + lane_ids
                vals = plsc.load_gather(
                    slab_v.at[t], [d_idx, jnp.full((lanes,), 0, jnp.int32) + lane_t]
                )
                rows_v[t, pl.ds(c * lanes, lanes)] = vals
        pltpu.sync_copy(rows_v, out_hbm.at[pl.ds(base, b_per_w)])

    return gather_kernel(table_t, ids)


def _matmul_body(h_ref, w_ref, out_ref):
    out_ref[...] = lax.dot_general(
        h_ref[...], w_ref[...],
        dimension_numbers=(((1,), (0,)), ((), ())),
        preferred_element_type=jnp.float32,
    )


def _tc_logits(h, head_w_t):
    grid = pl.cdiv(VOCAB, BLK)
    return pl.pallas_call(
        _matmul_body,
        grid=(grid,),
        in_specs=[
            pl.BlockSpec((TOKENS, HIDDEN), lambda i: (0, 0)),
            pl.BlockSpec((HIDDEN, BLK), lambda i: (0, i)),
        ],
        out_specs=pl.BlockSpec((TOKENS, BLK), lambda i: (0, i)),
        out_shape=jax.ShapeDtypeStruct((TOKENS, VOCAB), jnp.float32),
    )(h, head_w_t)


def kernel(input_ids, embed, head_w):
    b, l = input_ids.shape
    ids_flat = input_ids.reshape(-1).astype(jnp.int32)
    h = _sc_gather(embed.T, ids_flat)
    logits = _tc_logits(h, head_w.T)
    return logits.reshape(b, l, VOCAB)
